# baseline (speedup 1.0000x reference)
import jax
import jax.numpy as jnp
from jax.experimental import pallas as pl
from jax.experimental.pallas import tpu as pltpu


def _softmax_matmul_block(x, items):
    m = jnp.max(x, axis=-1, keepdims=True)
    e = jnp.exp(x - m)
    s = jnp.sum(e, axis=-1, keepdims=True)
    acc = jnp.dot(e, items, preferred_element_type=jnp.float32)
    return acc / s


def _fused_softmax_matmul(sel0_ref, sel1_ref, items_ref, out0_ref, out1_ref):
    items = items_ref[...]
    out0_ref[...] = _softmax_matmul_block(sel0_ref[...], items)
    out1_ref[...] = _softmax_matmul_block(sel1_ref[...], items)


def kernel(selections, items):
    batch, n_items = selections.shape
    n_items2, n_samples = items.shape
    assert n_items == n_items2
    tile_b = 1024
    grid = (batch // (2 * tile_b),)
    out0, out1 = pl.pallas_call(
        _fused_softmax_matmul,
        grid=grid,
        in_specs=[
            pl.BlockSpec((tile_b, n_items), lambda i: (2 * i, 0)),
            pl.BlockSpec((tile_b, n_items), lambda i: (2 * i + 1, 0)),
            pl.BlockSpec((n_items, n_samples), lambda i: (0, 0)),
        ],
        out_specs=[
            pl.BlockSpec((tile_b, n_samples), lambda i: (2 * i, 0)),
            pl.BlockSpec((tile_b, n_samples), lambda i: (2 * i + 1, 0)),
        ],
        out_shape=[
            jax.ShapeDtypeStruct((batch, n_samples), jnp.float32),
            jax.ShapeDtypeStruct((batch, n_samples), jnp.float32),
        ],
        compiler_params=pltpu.CompilerParams(
            dimension_semantics=("arbitrary",),
        ),
    )(selections, selections, items)
    idx = jax.lax.broadcasted_iota(jnp.int32, (batch, 1), 0) // tile_b
    return jnp.where(idx % 2 == 0, out0, out1)


# speedup vs baseline: 1.0124x; 1.0124x over previous
import jax
import jax.numpy as jnp
from jax.experimental import pallas as pl
from jax.experimental.pallas import tpu as pltpu

_TILE_B = 512


def _fused_softmax_matmul(sel_ref, items_ref, out_ref):
    x = sel_ref[...]
    m = jnp.max(x, axis=-1, keepdims=True)
    e = jnp.exp(x - m)
    s = jnp.sum(e, axis=-1, keepdims=True)
    acc = jnp.dot(e, items_ref[...], preferred_element_type=jnp.float32)
    out_ref[...] = acc / s


def kernel(selections, items):
    batch, n_items = selections.shape
    n_items2, n_samples = items.shape
    assert n_items == n_items2
    grid = (batch // _TILE_B,)
    mesh = pltpu.create_tensorcore_mesh("core")

    def body(sel_hbm, items_hbm, out_hbm):
        pltpu.emit_pipeline(
            _fused_softmax_matmul,
            grid=grid,
            in_specs=[
                pl.BlockSpec((_TILE_B, n_items), lambda i: (i, 0)),
                pl.BlockSpec((n_items, n_samples), lambda i: (0, 0)),
            ],
            out_specs=[pl.BlockSpec((_TILE_B, n_samples), lambda i: (i, 0))],
            core_axis_name="core",
            dimension_semantics=(pltpu.PARALLEL,),
        )(sel_hbm, items_hbm, out_hbm)

    run = pl.kernel(
        body,
        out_type=jax.ShapeDtypeStruct((batch, n_samples), jnp.float32),
        mesh=mesh,
    )
    return run(selections, items)


# speedup vs baseline: 1.0364x; 1.0236x over previous
import jax
import jax.numpy as jnp
from jax.experimental import pallas as pl
from jax.experimental.pallas import tpu as pltpu

_TILE_B = 512
_NBUF = 8


def _fused_softmax_matmul(sel_hbm, items_ref, out_ref, buf, sems):
    n_chunks = sel_hbm.shape[0] // _TILE_B
    items = items_ref[...]

    def copy_in(i, slot):
        return pltpu.make_async_copy(
            sel_hbm.at[pl.ds(i * _TILE_B, _TILE_B), :],
            buf.at[slot],
            sems.at[slot],
        )

    for j in range(min(_NBUF, n_chunks)):
        copy_in(j, j).start()

    for i in range(n_chunks):
        slot = i % _NBUF
        copy_in(i, slot).wait()
        x = buf[slot]
        m = jnp.max(x, axis=-1, keepdims=True)
        e = jnp.exp(x - m)
        s = jnp.sum(e, axis=-1, keepdims=True)
        acc = jnp.dot(e, items, preferred_element_type=jnp.float32)
        out_ref[pl.ds(i * _TILE_B, _TILE_B), :] = acc / s
        if i + _NBUF < n_chunks:
            copy_in(i + _NBUF, slot).start()


def kernel(selections, items):
    batch, n_items = selections.shape
    n_items2, n_samples = items.shape
    assert n_items == n_items2
    return pl.pallas_call(
        _fused_softmax_matmul,
        in_specs=[
            pl.BlockSpec(memory_space=pl.ANY),
            pl.BlockSpec(memory_space=pltpu.MemorySpace.VMEM),
        ],
        out_specs=pl.BlockSpec(memory_space=pltpu.MemorySpace.VMEM),
        out_shape=jax.ShapeDtypeStruct((batch, n_samples), jnp.float32),
        scratch_shapes=[
            pltpu.VMEM((_NBUF, _TILE_B, n_items), jnp.float32),
            pltpu.SemaphoreType.DMA((_NBUF,)),
        ],
    )(selections, items)
